# Initial kernel scaffold; baseline (speedup 1.0000x reference)
import jax
import jax.numpy as jnp
from jax.experimental import pallas as pl


def kernel(x, table):
    raise NotImplementedError("write your pallas kernel here")


# speedup vs baseline: 3.1344x; 3.1344x over previous
import functools

import jax
import jax.numpy as jnp
from jax import lax
from jax.experimental import pallas as pl
from jax.experimental.pallas import tpu as pltpu
from jax.experimental.pallas import tpu_sc as plsc

_IN_DIM = 101
_D = 64
_BATCH = 16384
_HIST = 200
_MAX_NORM = 1.0

_B = _BATCH * _HIST
_IDXW = 128
_NROWS = _B // _IDXW

_NC = 2
_NS = 16
_NW = _NC * _NS
_ROWS_PW = _NROWS // _NW

_NSUB = 4
_C = _NSUB * _IDXW
_G = _ROWS_PW // _NSUB


def _norm_body(tab_ref, out_ref):
    t = tab_ref[...]
    norms = jnp.sqrt(jnp.sum(t * t, axis=1, keepdims=True))
    scale = jnp.where(norms > _MAX_NORM, _MAX_NORM / (norms + 1e-7), 1.0)
    out_ref[...] = t * scale


def _normalize_table(table):
    return pl.pallas_call(
        _norm_body,
        out_shape=jax.ShapeDtypeStruct((_IN_DIM, _D), jnp.float32),
    )(table)


def _gather_body(tab_hbm, idx_hbm, out_hbm, idx_v, rows_v, gsem):
    wid = lax.axis_index("s") * _NC + lax.axis_index("c")
    row_base = wid * _ROWS_PW
    out_base = wid * _ROWS_PW * _IDXW

    def chunk(g, carry):
        pltpu.sync_copy(idx_hbm.at[pl.ds(row_base + g * _NSUB, _NSUB)], idx_v)
        cps = [
            pltpu.async_copy(
                tab_hbm.at[idx_v.at[j]],
                rows_v.at[pl.ds(j * _IDXW, _IDXW)],
                gsem,
            )
            for j in range(_NSUB)
        ]
        for cp in cps:
            cp.wait()
        pltpu.sync_copy(rows_v, out_hbm.at[pl.ds(out_base + g * _C, _C)])
        return carry

    lax.fori_loop(0, _G, chunk, 0)


@functools.partial(
    pl.kernel,
    out_type=jax.ShapeDtypeStruct((_B, _D), jnp.float32),
    mesh=plsc.VectorSubcoreMesh(core_axis_name="c", subcore_axis_name="s"),
    scratch_types=[
        pltpu.VMEM((_NSUB, _IDXW), jnp.int32),
        pltpu.VMEM((_C, _D), jnp.float32),
        pltpu.SemaphoreType.DMA,
    ],
    compiler_params=pltpu.CompilerParams(use_tc_tiling_on_sc=False),
)
def _sc_gather(tab_hbm, idx_hbm, out_hbm, idx_v, rows_v, gsem):
    _gather_body(tab_hbm, idx_hbm, out_hbm, idx_v, rows_v, gsem)


def kernel(x, table):
    norm_tab = _normalize_table(table)
    idx = x.reshape(_NROWS, _IDXW)
    flat = _sc_gather(norm_tab, idx)
    return flat.reshape(_BATCH, _HIST, _D)
